# baseline (speedup 1.0000x reference)
import functools

import jax
import jax.numpy as jnp
from jax import lax
from jax.experimental import pallas as pl
from jax.experimental.pallas import tpu as pltpu
from jax.experimental.pallas import tpu_sc as plsc

_N = 8192
_C = 1000
_CP = 1008
_BLOCK = 4096
_NW = 32
_CHUNK = _N // _NW


@functools.partial(
    pl.kernel,
    mesh=plsc.VectorSubcoreMesh(core_axis_name="c", subcore_axis_name="s"),
    compiler_params=pltpu.CompilerParams(needs_layout_passes=False),
    out_type=jax.ShapeDtypeStruct((_NW, _CP), jnp.float32),
    scratch_types=[
        pltpu.VMEM((_CHUNK,), jnp.int32),
        pltpu.VMEM((16 * _CP,), jnp.float32),
        pltpu.VMEM((_CP,), jnp.float32),
    ],
)
def _sc_hist(t_hbm, hist_hbm, t_v, bank_v, row_v):
    wid = lax.axis_index("s") * 2 + lax.axis_index("c")
    base = wid * _CHUNK
    pltpu.sync_copy(t_hbm.at[pl.ds(base, _CHUNK)], t_v)

    lane = lax.iota(jnp.int32, 16)
    zeros16 = jnp.zeros((16,), jnp.float32)
    ones16 = jnp.ones((16,), jnp.float32)

    def _zero(i, _):
        for u in range(4):
            bank_v[pl.ds(i * 64 + u * 16, 16)] = zeros16
        return 0

    lax.fori_loop(0, (16 * _CP) // 64, _zero, 0)

    for j in range(_CHUNK // 16):
        t16 = t_v[pl.ds(j * 16, 16)]
        plsc.addupdate_scatter(bank_v, [t16 + lane * _CP], ones16)

    def _fold(cb, _):
        acc = bank_v[pl.ds(cb * 16, 16)]
        for l in range(1, 16):
            acc = acc + bank_v[pl.ds(l * _CP + cb * 16, 16)]
        row_v[pl.ds(cb * 16, 16)] = acc
        return 0

    lax.fori_loop(0, _CP // 16, _fold, 0)
    pltpu.sync_copy(row_v, hist_hbm.at[wid])


def _tc_main(xt_ref, t_ref, out_ref, focal_ref):
    step = pl.program_id(0)
    nsteps = pl.num_programs(0)

    @pl.when(step == 0)
    def _init():
        focal_ref[0, 0] = 0.0

    x = xt_ref[...]
    m = jnp.max(x, axis=0, keepdims=True)
    z = jnp.sum(jnp.exp(x - m), axis=0, keepdims=True)
    row = jax.lax.broadcasted_iota(jnp.int32, x.shape, 0)
    mask = row == t_ref[...]
    xt = jnp.sum(jnp.where(mask, x, 0.0), axis=0, keepdims=True)
    logp = xt - m - jnp.log(z)
    p = jnp.exp(logp)
    one_m_p = 1.0 - p
    focal_ref[0, 0] += jnp.sum(one_m_p * one_m_p * (-logp))

    @pl.when(step == nsteps - 1)
    def _finish():
        out_ref[...] = jnp.full((1, 1), focal_ref[0, 0], dtype=jnp.float32)


def _tc_combine(hist_ref, focal_ref, out_ref):
    counts = jnp.sum(hist_ref[...], axis=0, keepdims=True)
    counts = counts[:, :_C]
    n = jnp.float32(_N)
    s_alpha = n - jnp.sum(counts * counts) / (10.0 * n)
    loss = s_alpha * focal_ref[0, 0] / (n * n)
    out_ref[...] = jnp.full((1, 1), loss, dtype=jnp.float32)


@jax.jit
def kernel(inputs, targets):
    hists = _sc_hist(targets)
    xT = inputs.T
    t2d = targets.reshape(1, _N)
    grid = _N // _BLOCK
    focal = pl.pallas_call(
        _tc_main,
        grid=(grid,),
        in_specs=[
            pl.BlockSpec((_C, _BLOCK), lambda i: (0, i)),
            pl.BlockSpec((1, _BLOCK), lambda i: (0, i)),
        ],
        out_specs=pl.BlockSpec((1, 1), lambda i: (0, 0)),
        out_shape=jax.ShapeDtypeStruct((1, 1), jnp.float32),
        scratch_shapes=[
            pltpu.SMEM((1, 1), jnp.float32),
        ],
    )(xT, t2d)
    out = pl.pallas_call(
        _tc_combine,
        out_shape=jax.ShapeDtypeStruct((1, 1), jnp.float32),
    )(hists, focal)
    return out[0, 0]


# speedup vs baseline: 1.1349x; 1.1349x over previous
import functools

import jax
import jax.numpy as jnp
from jax import lax
from jax.experimental import pallas as pl
from jax.experimental.pallas import tpu as pltpu
from jax.experimental.pallas import tpu_sc as plsc

_N = 8192
_C = 1000
_CP = 1008
_BLOCK = 2048
_NW = 32
_CHUNK = _N // _NW


@functools.partial(
    pl.kernel,
    mesh=plsc.VectorSubcoreMesh(core_axis_name="c", subcore_axis_name="s"),
    compiler_params=pltpu.CompilerParams(needs_layout_passes=False),
    out_type=jax.ShapeDtypeStruct((_NW, _CP), jnp.float32),
    scratch_types=[
        pltpu.VMEM((_CHUNK,), jnp.int32),
        pltpu.VMEM((16 * _CP,), jnp.float32),
        pltpu.VMEM((_CP,), jnp.float32),
    ],
)
def _sc_hist(t_hbm, hist_hbm, t_v, bank_v, row_v):
    wid = lax.axis_index("s") * 2 + lax.axis_index("c")
    base = wid * _CHUNK
    pltpu.sync_copy(t_hbm.at[pl.ds(base, _CHUNK)], t_v)

    lane = lax.iota(jnp.int32, 16)
    zeros16 = jnp.zeros((16,), jnp.float32)
    ones16 = jnp.ones((16,), jnp.float32)

    def _zero(i, _):
        for u in range(4):
            bank_v[pl.ds(i * 64 + u * 16, 16)] = zeros16
        return 0

    lax.fori_loop(0, (16 * _CP) // 64, _zero, 0)

    for j in range(_CHUNK // 16):
        t16 = t_v[pl.ds(j * 16, 16)]
        plsc.addupdate_scatter(bank_v, [t16 + lane * _CP], ones16)

    def _fold(cb, _):
        acc = bank_v[pl.ds(cb * 16, 16)]
        for l in range(1, 16):
            acc = acc + bank_v[pl.ds(l * _CP + cb * 16, 16)]
        row_v[pl.ds(cb * 16, 16)] = acc
        return 0

    lax.fori_loop(0, _CP // 16, _fold, 0)
    pltpu.sync_copy(row_v, hist_hbm.at[wid])


def _tc_main(xt_ref, t_ref, out_ref, focal_ref):
    step = pl.program_id(0)
    nsteps = pl.num_programs(0)

    @pl.when(step == 0)
    def _init():
        focal_ref[0, 0] = 0.0

    x = xt_ref[...]
    m = jnp.max(x, axis=0, keepdims=True)
    e = jnp.exp(x - m)
    row = jax.lax.broadcasted_iota(jnp.int32, x.shape, 0)
    mask = row == t_ref[...]
    mx = jnp.where(mask, x, 0.0)
    ones_row = jnp.ones((1, _C), jnp.float32)
    dn = (((1,), (0,)), ((), ()))
    z = jax.lax.dot_general(ones_row, e, dn,
                            precision=jax.lax.Precision.DEFAULT,
                            preferred_element_type=jnp.float32)
    xt = jax.lax.dot_general(ones_row, mx, dn,
                             precision=jax.lax.Precision.DEFAULT,
                             preferred_element_type=jnp.float32)
    logp = xt - m - jnp.log(z)
    p = jnp.exp(logp)
    one_m_p = 1.0 - p
    focal_ref[0, 0] += jnp.sum(one_m_p * one_m_p * (-logp))

    @pl.when(step == nsteps - 1)
    def _finish():
        out_ref[...] = jnp.full((1, 1), focal_ref[0, 0], dtype=jnp.float32)


def _tc_combine(hist_ref, focal_ref, out_ref):
    counts = jnp.sum(hist_ref[...], axis=0, keepdims=True)
    counts = counts[:, :_C]
    n = jnp.float32(_N)
    s_alpha = n - jnp.sum(counts * counts) / (10.0 * n)
    loss = s_alpha * focal_ref[0, 0] / (n * n)
    out_ref[...] = jnp.full((1, 1), loss, dtype=jnp.float32)


@jax.jit
def kernel(inputs, targets):
    hists = _sc_hist(targets)
    xT = inputs.T
    t2d = targets.reshape(1, _N)
    grid = _N // _BLOCK
    focal = pl.pallas_call(
        _tc_main,
        grid=(grid,),
        in_specs=[
            pl.BlockSpec((_C, _BLOCK), lambda i: (0, i)),
            pl.BlockSpec((1, _BLOCK), lambda i: (0, i)),
        ],
        out_specs=pl.BlockSpec((1, 1), lambda i: (0, 0)),
        out_shape=jax.ShapeDtypeStruct((1, 1), jnp.float32),
        scratch_shapes=[
            pltpu.SMEM((1, 1), jnp.float32),
        ],
    )(xT, t2d)
    out = pl.pallas_call(
        _tc_combine,
        out_shape=jax.ShapeDtypeStruct((1, 1), jnp.float32),
    )(hists, focal)
    return out[0, 0]


# speedup vs baseline: 1.1401x; 1.0045x over previous
import functools

import jax
import jax.numpy as jnp
from jax import lax
from jax.experimental import pallas as pl
from jax.experimental.pallas import tpu as pltpu
from jax.experimental.pallas import tpu_sc as plsc

_N = 8192
_C = 1000
_CP = 1008
_BLOCK = 2048
_NW = 32
_CHUNK = _N // _NW


@functools.partial(
    pl.kernel,
    mesh=plsc.VectorSubcoreMesh(core_axis_name="c", subcore_axis_name="s"),
    compiler_params=pltpu.CompilerParams(needs_layout_passes=False),
    out_type=jax.ShapeDtypeStruct((_NW, _CP), jnp.float32),
    scratch_types=[
        pltpu.VMEM((_CHUNK,), jnp.int32),
        pltpu.VMEM((16 * _CP,), jnp.float32),
        pltpu.VMEM((_CP,), jnp.float32),
    ],
)
def _sc_hist(t_hbm, hist_hbm, t_v, bank_v, row_v):
    wid = lax.axis_index("s") * 2 + lax.axis_index("c")
    base = wid * _CHUNK
    pltpu.sync_copy(t_hbm.at[pl.ds(base, _CHUNK)], t_v)

    lane = lax.iota(jnp.int32, 16)
    zeros16 = jnp.zeros((16,), jnp.float32)
    ones16 = jnp.ones((16,), jnp.float32)

    def _zero(i, _):
        for u in range(4):
            bank_v[pl.ds(i * 64 + u * 16, 16)] = zeros16
        return 0

    lax.fori_loop(0, (16 * _CP) // 64, _zero, 0)

    for j in range(_CHUNK // 16):
        t16 = t_v[pl.ds(j * 16, 16)]
        plsc.addupdate_scatter(bank_v, [t16 + lane * _CP], ones16)

    def _fold(cb, _):
        acc = bank_v[pl.ds(cb * 16, 16)]
        for l in range(1, 16):
            acc = acc + bank_v[pl.ds(l * _CP + cb * 16, 16)]
        row_v[pl.ds(cb * 16, 16)] = acc
        return 0

    lax.fori_loop(0, _CP // 16, _fold, 0)
    pltpu.sync_copy(row_v, hist_hbm.at[wid])


def _tc_main(xt_ref, t_ref, out_ref, focal_ref):
    step = pl.program_id(0)
    nsteps = pl.num_programs(0)

    @pl.when(step == 0)
    def _init():
        focal_ref[0, 0] = 0.0

    x = xt_ref[...]
    m = jnp.max(x, axis=0, keepdims=True)
    eb = jnp.exp(x - m).astype(jnp.bfloat16)
    row = jax.lax.broadcasted_iota(jnp.int32, x.shape, 0)
    mask = row == t_ref[...]
    meb = jnp.where(mask, eb, jnp.bfloat16(0.0))
    ones_row = jnp.ones((1, _C), jnp.bfloat16)
    dn = (((1,), (0,)), ((), ()))
    z = jax.lax.dot_general(ones_row, eb, dn,
                            preferred_element_type=jnp.float32)
    pexp = jax.lax.dot_general(ones_row, meb, dn,
                               preferred_element_type=jnp.float32)
    logp = jnp.log(pexp) - jnp.log(z)
    p = jnp.exp(logp)
    one_m_p = 1.0 - p
    focal_ref[0, 0] += jnp.sum(one_m_p * one_m_p * (-logp))

    @pl.when(step == nsteps - 1)
    def _finish():
        out_ref[...] = jnp.full((1, 1), focal_ref[0, 0], dtype=jnp.float32)


def _tc_combine(hist_ref, focal_ref, out_ref):
    counts = jnp.sum(hist_ref[...], axis=0, keepdims=True)
    counts = counts[:, :_C]
    n = jnp.float32(_N)
    s_alpha = n - jnp.sum(counts * counts) / (10.0 * n)
    loss = s_alpha * focal_ref[0, 0] / (n * n)
    out_ref[...] = jnp.full((1, 1), loss, dtype=jnp.float32)


@jax.jit
def kernel(inputs, targets):
    hists = _sc_hist(targets)
    xT = inputs.T
    t2d = targets.reshape(1, _N)
    grid = _N // _BLOCK
    focal = pl.pallas_call(
        _tc_main,
        grid=(grid,),
        in_specs=[
            pl.BlockSpec((_C, _BLOCK), lambda i: (0, i)),
            pl.BlockSpec((1, _BLOCK), lambda i: (0, i)),
        ],
        out_specs=pl.BlockSpec((1, 1), lambda i: (0, 0)),
        out_shape=jax.ShapeDtypeStruct((1, 1), jnp.float32),
        scratch_shapes=[
            pltpu.SMEM((1, 1), jnp.float32),
        ],
    )(xT, t2d)
    out = pl.pallas_call(
        _tc_combine,
        out_shape=jax.ShapeDtypeStruct((1, 1), jnp.float32),
    )(hists, focal)
    return out[0, 0]
